# baseline (speedup 1.0000x reference)
import functools

import jax
import jax.numpy as jnp
from jax import lax
from jax.experimental import pallas as pl
from jax.experimental.pallas import tpu as pltpu
from jax.experimental.pallas import tpu_sc as plsc

_N = 10000
_E = 160000
_G = 64
_DIN = 256
_DH = 512
_DOUT = 128

_F = 128
_NT = 16
_NC = 2
_S0 = 632
_SL = _N - (_NT - 1) * _S0
_K = 125
_RPT = (_E // _K) // _NT
_WIN = _RPT // 2
_KH = 125
_RPW = (_E // _KH) // (_NC * _NT)
_BN = 1000

_mesh = plsc.VectorSubcoreMesh(core_axis_name="c", subcore_axis_name="s")


def _striped(t, fn):
    @pl.when(t < _NT - 1)
    def _main():
        fn(t * _S0, _S0)

    @pl.when(t == _NT - 1)
    def _last():
        fn((_NT - 1) * _S0, _SL)


def _hist_body(dst2_hbm, out0_hbm, out1_hbm, hist, zbuf, obuf, hidx):
    c = lax.axis_index("c")
    t = lax.axis_index("s")

    def _zb(i, carry):
        zbuf[pl.ds(i * 16, 16)] = jnp.zeros((16,), jnp.float32)
        return carry

    lax.fori_loop(0, _S0 // 8 // 2 + 1, _zb, 0)

    def _ob(i, carry):
        obuf[pl.ds(i * 16, 16)] = jnp.ones((16,), jnp.float32)
        return carry

    lax.fori_loop(0, 8, _ob, 0)

    def _zero(row0, nrows):
        pltpu.sync_copy(zbuf.at[pl.ds(0, nrows)], hist.at[pl.ds(row0, nrows)])

    _striped(t, _zero)
    plsc.subcore_barrier()

    w = c * _NT + t
    pltpu.sync_copy(dst2_hbm.at[pl.ds(w * _RPW, _RPW)], hidx)

    def _body(j, carry):
        pltpu.sync_copy(obuf.at[pl.ds(0, _KH)], hist.at[hidx.at[j]], add=True)
        return carry

    lax.fori_loop(0, _RPW, _body, 0)
    plsc.subcore_barrier()

    @pl.when(c == 0)
    def _d0():
        def _drain(row0, nrows):
            pltpu.sync_copy(hist.at[pl.ds(row0, nrows)],
                            zbuf.at[pl.ds(0, nrows)])
            pltpu.sync_copy(zbuf.at[pl.ds(0, nrows)],
                            out0_hbm.at[pl.ds(row0, nrows)])
        _striped(t, _drain)

    @pl.when(c == 1)
    def _d1():
        def _drain(row0, nrows):
            pltpu.sync_copy(hist.at[pl.ds(row0, nrows)],
                            zbuf.at[pl.ds(0, nrows)])
            pltpu.sync_copy(zbuf.at[pl.ds(0, nrows)],
                            out1_hbm.at[pl.ds(row0, nrows)])
        _striped(t, _drain)


_hist = functools.partial(
    pl.kernel,
    out_type=[jax.ShapeDtypeStruct((_N,), jnp.float32),
              jax.ShapeDtypeStruct((_N,), jnp.float32)],
    mesh=_mesh,
    scratch_types=[
        pltpu.VMEM_SHARED((_N,), jnp.float32),
        pltpu.VMEM((_S0 + 8, ), jnp.float32),
        pltpu.VMEM((128,), jnp.float32),
        pltpu.VMEM((_RPW, _KH), jnp.int32),
    ],
)(_hist_body)


def _make_agg(num_slices):
    spc = num_slices // _NC

    def body(xs_hbm, src2_hbm, dst2_hbm, out_hbm,
             acc, sidx, didx, rb0, rb1, sem0, sem1, sem2, sem3):
        c = lax.axis_index("c")
        t = lax.axis_index("s")
        bufs = (rb0, rb1)
        sems = (sem0, sem1)
        ssems = (sem2, sem3)

        for cv in range(_NC):
            @pl.when(c == cv)
            def _core():
                for j in range(spc):
                    s = cv * spc + j
                    tbl = xs_hbm.at[s]

                    def _init(row0, nrows):
                        pltpu.sync_copy(tbl.at[pl.ds(row0, nrows)],
                                        acc.at[pl.ds(row0, nrows)])
                    _striped(t, _init)
                    plsc.subcore_barrier()

                    def _gstart(r, b):
                        pltpu.make_async_copy(tbl.at[sidx.at[r]],
                                              bufs[b], sems[b]).start()

                    def _gwait(b):
                        pltpu.make_async_copy(tbl.at[sidx.at[0]],
                                              bufs[b], sems[b]).wait()

                    def _sstart(r, b):
                        pltpu.async_copy(bufs[b], acc.at[didx.at[r]],
                                         ssems[b], add=True)

                    def _swait(b):
                        pltpu.make_async_copy(bufs[b], acc.at[didx.at[0]],
                                              ssems[b]).wait()

                    for g in range(_RPT // _WIN):
                        base = t * _RPT + g * _WIN
                        pltpu.sync_copy(src2_hbm.at[pl.ds(base, _WIN)], sidx)
                        pltpu.sync_copy(dst2_hbm.at[pl.ds(base, _WIN)], didx)
                        _gstart(0, 0)
                        _gstart(1, 1)

                        def _pair(p, carry):
                            r0 = 2 * p
                            r1 = r0 + 1

                            _gwait(0)
                            _sstart(r0, 0)
                            _gwait(1)
                            _sstart(r1, 1)

                            @pl.when(r0 + 2 < _WIN)
                            def _g0():
                                _swait(0)
                                _gstart(r0 + 2, 0)

                            @pl.when(r1 + 2 < _WIN)
                            def _g1():
                                _swait(1)
                                _gstart(r1 + 2, 1)

                            return carry

                        lax.fori_loop(0, _WIN // 2, _pair, 0)
                        _swait(0)
                        _swait(1)

                    plsc.subcore_barrier()

                    def _drain(row0, nrows):
                        pltpu.sync_copy(acc.at[pl.ds(row0, nrows)],
                                        out_hbm.at[s].at[pl.ds(row0, nrows)])
                    _striped(t, _drain)

    return functools.partial(
        pl.kernel,
        out_type=jax.ShapeDtypeStruct((num_slices, _N, _F), jnp.float32),
        mesh=_mesh,
        scratch_types=[
            pltpu.VMEM_SHARED((_N, _F), jnp.float32),
            pltpu.VMEM((_WIN, _K), jnp.int32),
            pltpu.VMEM((_WIN, _K), jnp.int32),
            pltpu.VMEM((_K, _F), jnp.float32),
            pltpu.VMEM((_K, _F), jnp.float32),
            pltpu.SemaphoreType.DMA,
            pltpu.SemaphoreType.DMA,
            pltpu.SemaphoreType.DMA,
            pltpu.SemaphoreType.DMA,
        ],
    )(body)


_agg_k2 = _make_agg(2)
_agg_k4 = _make_agg(4)


def _prep_body(x_ref, d0_ref, d1_ref, dinv_ref, xs_ref):
    deg = (d0_ref[...] + d1_ref[...]).reshape(_BN) + 1.0
    di = lax.rsqrt(deg)
    dinv_ref[...] = di.reshape(1, 1, _BN)
    xsb = x_ref[...] * di[:, None]
    xs_ref[0] = xsb[:, :_F]
    xs_ref[1] = xsb[:, _F:]


def _prep(x, d0, d1):
    return pl.pallas_call(
        _prep_body,
        grid=(_N // _BN,),
        in_specs=[
            pl.BlockSpec((_BN, _DIN), lambda i: (i, 0)),
            pl.BlockSpec((1, 1, _BN), lambda i: (i, 0, 0)),
            pl.BlockSpec((1, 1, _BN), lambda i: (i, 0, 0)),
        ],
        out_specs=[
            pl.BlockSpec((1, 1, _BN), lambda i: (i, 0, 0)),
            pl.BlockSpec((2, _BN, _F), lambda i: (0, i, 0)),
        ],
        out_shape=[
            jax.ShapeDtypeStruct((_N // _BN, 1, _BN), jnp.float32),
            jax.ShapeDtypeStruct((2, _N, _F), jnp.float32),
        ],
    )(x, d0, d1)


def _layer_body(num_slices, scale_out, agg_ref, dinv_ref, w_ref, b_ref, out_ref):
    di = dinv_ref[...].reshape(_BN)
    h = jnp.zeros((_BN, _DH), jnp.float32)
    for s in range(num_slices):
        a = agg_ref[s] * di[:, None]
        h = h + jnp.dot(a, w_ref[s], preferred_element_type=jnp.float32)
    h = jnp.maximum(h + b_ref[...][None, :], 0.0)
    if scale_out:
        h = h * di[:, None]
        for s in range(_DH // _F):
            out_ref[s] = h[:, s * _F:(s + 1) * _F]
    else:
        out_ref[...] = h


def _layer(agg, dinv, w, b, scale_out):
    num_slices = w.shape[0]
    if scale_out:
        out_spec = pl.BlockSpec((_DH // _F, _BN, _F), lambda i: (0, i, 0))
        out_shape = jax.ShapeDtypeStruct((_DH // _F, _N, _F), jnp.float32)
    else:
        out_spec = pl.BlockSpec((_BN, _DH), lambda i: (i, 0))
        out_shape = jax.ShapeDtypeStruct((_N, _DH), jnp.float32)
    return pl.pallas_call(
        functools.partial(_layer_body, num_slices, scale_out),
        grid=(_N // _BN,),
        in_specs=[
            pl.BlockSpec((num_slices, _BN, _F), lambda i: (0, i, 0)),
            pl.BlockSpec((1, 1, _BN), lambda i: (i, 0, 0)),
            pl.BlockSpec((num_slices, _F, _DH), lambda i: (0, 0, 0)),
            pl.BlockSpec((_DH,), lambda i: (0,)),
        ],
        out_specs=out_spec,
        out_shape=out_shape,
    )(agg, dinv, w, b)


def _pool_body(h2_ref, batch_ref, wout_ref, bout_ref, out_ref, acc, cnt):
    i = pl.program_id(0)

    @pl.when(i == 0)
    def _init():
        acc[...] = jnp.zeros_like(acc)
        cnt[...] = jnp.zeros_like(cnt)

    bt = batch_ref[...].reshape(_BN)
    gid = lax.broadcasted_iota(jnp.int32, (_G, _BN), 0)
    oh = (bt[None, :] == gid).astype(jnp.float32)
    acc[...] += jnp.dot(oh, h2_ref[...], preferred_element_type=jnp.float32)
    cnt[...] += jnp.sum(oh, axis=1, keepdims=True)

    @pl.when(i == _N // _BN - 1)
    def _fin():
        pooled = acc[...] / jnp.maximum(cnt[...], 1.0)
        out_ref[...] = (jnp.dot(pooled, wout_ref[...],
                                preferred_element_type=jnp.float32)
                        + bout_ref[...][None, :])


def _pool(h2, batch3, wout, bout):
    return pl.pallas_call(
        _pool_body,
        grid=(_N // _BN,),
        in_specs=[
            pl.BlockSpec((_BN, _DH), lambda i: (i, 0)),
            pl.BlockSpec((1, 1, _BN), lambda i: (i, 0, 0)),
            pl.BlockSpec((_DH, _DOUT), lambda i: (0, 0)),
            pl.BlockSpec((_DOUT,), lambda i: (0,)),
        ],
        out_specs=pl.BlockSpec((_G, _DOUT), lambda i: (0, 0)),
        out_shape=jax.ShapeDtypeStruct((_G, _DOUT), jnp.float32),
        scratch_shapes=[
            pltpu.VMEM((_G, _DH), jnp.float32),
            pltpu.VMEM((_G, 1), jnp.float32),
        ],
    )(h2, batch3, wout, bout)


def kernel(x, edge_index, batch, W1, b1, W2, b2, Wout, bout):
    src = edge_index[0]
    dst = edge_index[1]
    src2 = src.reshape(_E // _K, _K)
    dst2 = dst.reshape(_E // _K, _K)
    dst2h = dst.reshape(_E // _KH, _KH)
    batch3 = batch.reshape(_N // _BN, 1, _BN)

    deg0, deg1 = _hist(dst2h)
    d0 = deg0.reshape(_N // _BN, 1, _BN)
    d1 = deg1.reshape(_N // _BN, 1, _BN)
    dinv, xs = _prep(x, d0, d1)
    agg1 = _agg_k2(xs, src2, dst2)
    h1s = _layer(agg1, dinv, W1.reshape(2, _F, _DH), b1, scale_out=True)
    agg2 = _agg_k4(h1s, src2, dst2)
    h2 = _layer(agg2, dinv, W2.reshape(4, _F, _DH), b2, scale_out=False)
    return _pool(h2, batch3, Wout, bout)


# speedup vs baseline: 1.2503x; 1.2503x over previous
import functools

import jax
import jax.numpy as jnp
from jax import lax
from jax.experimental import pallas as pl
from jax.experimental.pallas import tpu as pltpu
from jax.experimental.pallas import tpu_sc as plsc

_N = 10000
_E = 160000
_G = 64
_DIN = 256
_DH = 512
_DOUT = 128

_F = 128
_NT = 16
_NC = 2
_S0 = 632
_SL = _N - (_NT - 1) * _S0
_K = 125
_RPT = (_E // _K) // _NT
_WIN = _RPT // 2
_KH = 125
_RPW = (_E // _KH) // (_NC * _NT)
_BN = 1000

_mesh = plsc.VectorSubcoreMesh(core_axis_name="c", subcore_axis_name="s")


def _striped(t, fn):
    @pl.when(t < _NT - 1)
    def _main():
        fn(t * _S0, _S0)

    @pl.when(t == _NT - 1)
    def _last():
        fn((_NT - 1) * _S0, _SL)


def _hist_body(dst2_hbm, out0_hbm, out1_hbm, hist, zbuf, obuf, hidx):
    c = lax.axis_index("c")
    t = lax.axis_index("s")

    def _zb(i, carry):
        zbuf[pl.ds(i * 16, 16)] = jnp.zeros((16,), jnp.float32)
        return carry

    lax.fori_loop(0, _S0 // 8 // 2 + 1, _zb, 0)

    def _ob(i, carry):
        obuf[pl.ds(i * 16, 16)] = jnp.ones((16,), jnp.float32)
        return carry

    lax.fori_loop(0, 8, _ob, 0)

    def _zero(row0, nrows):
        pltpu.sync_copy(zbuf.at[pl.ds(0, nrows)], hist.at[pl.ds(row0, nrows)])

    _striped(t, _zero)
    plsc.subcore_barrier()

    w = c * _NT + t
    pltpu.sync_copy(dst2_hbm.at[pl.ds(w * _RPW, _RPW)], hidx)

    def _body(j, carry):
        pltpu.sync_copy(obuf.at[pl.ds(0, _KH)], hist.at[hidx.at[j]], add=True)
        return carry

    lax.fori_loop(0, _RPW, _body, 0)
    plsc.subcore_barrier()

    @pl.when(c == 0)
    def _d0():
        def _drain(row0, nrows):
            pltpu.sync_copy(hist.at[pl.ds(row0, nrows)],
                            zbuf.at[pl.ds(0, nrows)])
            pltpu.sync_copy(zbuf.at[pl.ds(0, nrows)],
                            out0_hbm.at[pl.ds(row0, nrows)])
        _striped(t, _drain)

    @pl.when(c == 1)
    def _d1():
        def _drain(row0, nrows):
            pltpu.sync_copy(hist.at[pl.ds(row0, nrows)],
                            zbuf.at[pl.ds(0, nrows)])
            pltpu.sync_copy(zbuf.at[pl.ds(0, nrows)],
                            out1_hbm.at[pl.ds(row0, nrows)])
        _striped(t, _drain)


_hist = functools.partial(
    pl.kernel,
    out_type=[jax.ShapeDtypeStruct((_N,), jnp.float32),
              jax.ShapeDtypeStruct((_N,), jnp.float32)],
    mesh=_mesh,
    scratch_types=[
        pltpu.VMEM_SHARED((_N,), jnp.float32),
        pltpu.VMEM((_S0 + 8, ), jnp.float32),
        pltpu.VMEM((128,), jnp.float32),
        pltpu.VMEM((_RPW, _KH), jnp.int32),
    ],
)(_hist_body)


def _make_agg(num_slices):
    spc = num_slices // _NC

    def body(xs_hbm, src2_hbm, dst2_hbm, out_hbm,
             acc, sidx, didx, rb0, rb1, sem0, sem1):
        c = lax.axis_index("c")
        t = lax.axis_index("s")
        bufs = (rb0, rb1)
        sems = (sem0, sem1)

        for cv in range(_NC):
            @pl.when(c == cv)
            def _core():
                for j in range(spc):
                    s = cv * spc + j
                    tbl = xs_hbm.at[s]

                    def _init(row0, nrows):
                        pltpu.sync_copy(tbl.at[pl.ds(row0, nrows)],
                                        acc.at[pl.ds(row0, nrows)])
                    _striped(t, _init)
                    plsc.subcore_barrier()

                    def _gstart(r, b):
                        pltpu.make_async_copy(tbl.at[sidx.at[r]],
                                              bufs[b], sems[b]).start()

                    def _gwait(b):
                        pltpu.make_async_copy(tbl.at[sidx.at[0]],
                                              bufs[b], sems[b]).wait()

                    for g in range(_RPT // _WIN):
                        base = t * _RPT + g * _WIN
                        pltpu.sync_copy(src2_hbm.at[pl.ds(base, _WIN)], sidx)
                        pltpu.sync_copy(dst2_hbm.at[pl.ds(base, _WIN)], didx)
                        _gstart(0, 0)

                        def _pair(p, carry):
                            r0 = 2 * p
                            r1 = r0 + 1

                            _gstart(r1, 1)
                            _gwait(0)
                            pltpu.sync_copy(rb0, acc.at[didx.at[r0]],
                                            add=True)

                            @pl.when(r0 + 2 < _WIN)
                            def _g0():
                                _gstart(r0 + 2, 0)

                            _gwait(1)
                            pltpu.sync_copy(rb1, acc.at[didx.at[r1]],
                                            add=True)
                            return carry

                        lax.fori_loop(0, _WIN // 2, _pair, 0)

                    plsc.subcore_barrier()

                    def _drain(row0, nrows):
                        pltpu.sync_copy(acc.at[pl.ds(row0, nrows)],
                                        out_hbm.at[s].at[pl.ds(row0, nrows)])
                    _striped(t, _drain)

    return functools.partial(
        pl.kernel,
        out_type=jax.ShapeDtypeStruct((num_slices, _N, _F), jnp.float32),
        mesh=_mesh,
        scratch_types=[
            pltpu.VMEM_SHARED((_N, _F), jnp.float32),
            pltpu.VMEM((_WIN, _K), jnp.int32),
            pltpu.VMEM((_WIN, _K), jnp.int32),
            pltpu.VMEM((_K, _F), jnp.float32),
            pltpu.VMEM((_K, _F), jnp.float32),
            pltpu.SemaphoreType.DMA,
            pltpu.SemaphoreType.DMA,
        ],
    )(body)


_agg_k2 = _make_agg(2)
_agg_k4 = _make_agg(4)


def _prep_body(x_ref, d0_ref, d1_ref, dinv_ref, xs_ref):
    deg = (d0_ref[...] + d1_ref[...]).reshape(_BN) + 1.0
    di = lax.rsqrt(deg)
    dinv_ref[...] = di.reshape(1, 1, _BN)
    xsb = x_ref[...] * di[:, None]
    xs_ref[0] = xsb[:, :_F]
    xs_ref[1] = xsb[:, _F:]


def _prep(x, d0, d1):
    return pl.pallas_call(
        _prep_body,
        grid=(_N // _BN,),
        in_specs=[
            pl.BlockSpec((_BN, _DIN), lambda i: (i, 0)),
            pl.BlockSpec((1, 1, _BN), lambda i: (i, 0, 0)),
            pl.BlockSpec((1, 1, _BN), lambda i: (i, 0, 0)),
        ],
        out_specs=[
            pl.BlockSpec((1, 1, _BN), lambda i: (i, 0, 0)),
            pl.BlockSpec((2, _BN, _F), lambda i: (0, i, 0)),
        ],
        out_shape=[
            jax.ShapeDtypeStruct((_N // _BN, 1, _BN), jnp.float32),
            jax.ShapeDtypeStruct((2, _N, _F), jnp.float32),
        ],
    )(x, d0, d1)


def _layer_body(num_slices, scale_out, agg_ref, dinv_ref, w_ref, b_ref, out_ref):
    di = dinv_ref[...].reshape(_BN)
    h = jnp.zeros((_BN, _DH), jnp.float32)
    for s in range(num_slices):
        a = agg_ref[s] * di[:, None]
        h = h + jnp.dot(a, w_ref[s], preferred_element_type=jnp.float32)
    h = jnp.maximum(h + b_ref[...][None, :], 0.0)
    if scale_out:
        h = h * di[:, None]
        for s in range(_DH // _F):
            out_ref[s] = h[:, s * _F:(s + 1) * _F]
    else:
        out_ref[...] = h


def _layer(agg, dinv, w, b, scale_out):
    num_slices = w.shape[0]
    if scale_out:
        out_spec = pl.BlockSpec((_DH // _F, _BN, _F), lambda i: (0, i, 0))
        out_shape = jax.ShapeDtypeStruct((_DH // _F, _N, _F), jnp.float32)
    else:
        out_spec = pl.BlockSpec((_BN, _DH), lambda i: (i, 0))
        out_shape = jax.ShapeDtypeStruct((_N, _DH), jnp.float32)
    return pl.pallas_call(
        functools.partial(_layer_body, num_slices, scale_out),
        grid=(_N // _BN,),
        in_specs=[
            pl.BlockSpec((num_slices, _BN, _F), lambda i: (0, i, 0)),
            pl.BlockSpec((1, 1, _BN), lambda i: (i, 0, 0)),
            pl.BlockSpec((num_slices, _F, _DH), lambda i: (0, 0, 0)),
            pl.BlockSpec((_DH,), lambda i: (0,)),
        ],
        out_specs=out_spec,
        out_shape=out_shape,
    )(agg, dinv, w, b)


def _pool_body(h2_ref, batch_ref, wout_ref, bout_ref, out_ref, acc, cnt):
    i = pl.program_id(0)

    @pl.when(i == 0)
    def _init():
        acc[...] = jnp.zeros_like(acc)
        cnt[...] = jnp.zeros_like(cnt)

    bt = batch_ref[...].reshape(_BN)
    gid = lax.broadcasted_iota(jnp.int32, (_G, _BN), 0)
    oh = (bt[None, :] == gid).astype(jnp.float32)
    acc[...] += jnp.dot(oh, h2_ref[...], preferred_element_type=jnp.float32)
    cnt[...] += jnp.sum(oh, axis=1, keepdims=True)

    @pl.when(i == _N // _BN - 1)
    def _fin():
        pooled = acc[...] / jnp.maximum(cnt[...], 1.0)
        out_ref[...] = (jnp.dot(pooled, wout_ref[...],
                                preferred_element_type=jnp.float32)
                        + bout_ref[...][None, :])


def _pool(h2, batch3, wout, bout):
    return pl.pallas_call(
        _pool_body,
        grid=(_N // _BN,),
        in_specs=[
            pl.BlockSpec((_BN, _DH), lambda i: (i, 0)),
            pl.BlockSpec((1, 1, _BN), lambda i: (i, 0, 0)),
            pl.BlockSpec((_DH, _DOUT), lambda i: (0, 0)),
            pl.BlockSpec((_DOUT,), lambda i: (0,)),
        ],
        out_specs=pl.BlockSpec((_G, _DOUT), lambda i: (0, 0)),
        out_shape=jax.ShapeDtypeStruct((_G, _DOUT), jnp.float32),
        scratch_shapes=[
            pltpu.VMEM((_G, _DH), jnp.float32),
            pltpu.VMEM((_G, 1), jnp.float32),
        ],
    )(h2, batch3, wout, bout)


def kernel(x, edge_index, batch, W1, b1, W2, b2, Wout, bout):
    src = edge_index[0]
    dst = edge_index[1]
    src2 = src.reshape(_E // _K, _K)
    dst2 = dst.reshape(_E // _K, _K)
    dst2h = dst.reshape(_E // _KH, _KH)
    batch3 = batch.reshape(_N // _BN, 1, _BN)

    deg0, deg1 = _hist(dst2h)
    d0 = deg0.reshape(_N // _BN, 1, _BN)
    d1 = deg1.reshape(_N // _BN, 1, _BN)
    dinv, xs = _prep(x, d0, d1)
    agg1 = _agg_k2(xs, src2, dst2)
    h1s = _layer(agg1, dinv, W1.reshape(2, _F, _DH), b1, scale_out=True)
    agg2 = _agg_k4(h1s, src2, dst2)
    h2 = _layer(agg2, dinv, W2.reshape(4, _F, _DH), b2, scale_out=False)
    return _pool(h2, batch3, Wout, bout)


# speedup vs baseline: 1.2692x; 1.0152x over previous
import functools

import jax
import jax.numpy as jnp
from jax import lax
from jax.experimental import pallas as pl
from jax.experimental.pallas import tpu as pltpu
from jax.experimental.pallas import tpu_sc as plsc

_N = 10000
_E = 160000
_G = 64
_DIN = 256
_DH = 512
_DOUT = 128

_F = 128
_NT = 16
_NC = 2
_S0 = 624
_SL = _N - (_NT - 1) * _S0
_K = 125
_RPT = (_E // _K) // _NT
_WIN = _RPT // 2
_KH = 125
_RPW = (_E // _KH) // (_NC * _NT)
_BN = 1000

_mesh = plsc.VectorSubcoreMesh(core_axis_name="c", subcore_axis_name="s")


def _striped(t, fn):
    @pl.when(t < _NT - 1)
    def _main():
        fn(t * _S0, _S0)

    @pl.when(t == _NT - 1)
    def _last():
        fn((_NT - 1) * _S0, _SL)


def _hist_body(dst2_hbm, out0_hbm, out1_hbm, hist, zbuf, obuf, hidx):
    c = lax.axis_index("c")
    t = lax.axis_index("s")

    def _zb(i, carry):
        zbuf[pl.ds(i * 16, 16)] = jnp.zeros((16,), jnp.float32)
        return carry

    lax.fori_loop(0, _SL // 16, _zb, 0)

    def _ob(i, carry):
        obuf[pl.ds(i * 16, 16)] = jnp.ones((16,), jnp.float32)
        return carry

    lax.fori_loop(0, 8, _ob, 0)

    def _zero(row0, nrows):
        pltpu.sync_copy(zbuf.at[pl.ds(0, nrows)], hist.at[pl.ds(row0, nrows)])

    _striped(t, _zero)
    plsc.subcore_barrier()

    w = c * _NT + t
    pltpu.sync_copy(dst2_hbm.at[pl.ds(w * _RPW, _RPW)], hidx)

    def _body(j, carry):
        pltpu.sync_copy(obuf.at[pl.ds(0, _KH)], hist.at[hidx.at[j]], add=True)
        return carry

    lax.fori_loop(0, _RPW, _body, 0)
    plsc.subcore_barrier()

    @pl.when(c == 0)
    def _d0():
        def _drain(row0, nrows):
            pltpu.sync_copy(hist.at[pl.ds(row0, nrows)],
                            zbuf.at[pl.ds(0, nrows)])
            pltpu.sync_copy(zbuf.at[pl.ds(0, nrows)],
                            out0_hbm.at[pl.ds(row0, nrows)])
        _striped(t, _drain)

    @pl.when(c == 1)
    def _d1():
        def _drain(row0, nrows):
            pltpu.sync_copy(hist.at[pl.ds(row0, nrows)],
                            zbuf.at[pl.ds(0, nrows)])
            pltpu.sync_copy(zbuf.at[pl.ds(0, nrows)],
                            out1_hbm.at[pl.ds(row0, nrows)])
        _striped(t, _drain)


_hist = functools.partial(
    pl.kernel,
    out_type=[jax.ShapeDtypeStruct((_N,), jnp.float32),
              jax.ShapeDtypeStruct((_N,), jnp.float32)],
    mesh=_mesh,
    scratch_types=[
        pltpu.VMEM_SHARED((_N,), jnp.float32),
        pltpu.VMEM((_SL,), jnp.float32),
        pltpu.VMEM((128,), jnp.float32),
        pltpu.VMEM((_RPW, _KH), jnp.int32),
    ],
)(_hist_body)


def _make_agg(num_slices, dtype):
    spc = num_slices // _NC

    def body(xs_hbm, src2_hbm, dst2_hbm, out_hbm,
             acc, sidx, didx, rb0, rb1, sem0, sem1):
        c = lax.axis_index("c")
        t = lax.axis_index("s")
        bufs = (rb0, rb1)
        sems = (sem0, sem1)

        for cv in range(_NC):
            @pl.when(c == cv)
            def _core():
                for j in range(spc):
                    s = cv * spc + j
                    tbl = xs_hbm.at[s]

                    def _init(row0, nrows):
                        pltpu.sync_copy(tbl.at[pl.ds(row0, nrows)],
                                        acc.at[pl.ds(row0, nrows)])
                    _striped(t, _init)
                    plsc.subcore_barrier()

                    def _gstart(r, b):
                        pltpu.make_async_copy(tbl.at[sidx.at[r]],
                                              bufs[b], sems[b]).start()

                    def _gwait(b):
                        pltpu.make_async_copy(tbl.at[sidx.at[0]],
                                              bufs[b], sems[b]).wait()

                    for g in range(_RPT // _WIN):
                        base = t * _RPT + g * _WIN
                        pltpu.sync_copy(src2_hbm.at[pl.ds(base, _WIN)], sidx)
                        pltpu.sync_copy(dst2_hbm.at[pl.ds(base, _WIN)], didx)
                        _gstart(0, 0)

                        def _pair(p, carry):
                            r0 = 2 * p
                            r1 = r0 + 1

                            _gstart(r1, 1)
                            _gwait(0)
                            pltpu.sync_copy(rb0, acc.at[didx.at[r0]],
                                            add=True)

                            @pl.when(r0 + 2 < _WIN)
                            def _g0():
                                _gstart(r0 + 2, 0)

                            _gwait(1)
                            pltpu.sync_copy(rb1, acc.at[didx.at[r1]],
                                            add=True)
                            return carry

                        lax.fori_loop(0, _WIN // 2, _pair, 0)

                    plsc.subcore_barrier()

                    def _drain(row0, nrows):
                        pltpu.sync_copy(acc.at[pl.ds(row0, nrows)],
                                        out_hbm.at[s].at[pl.ds(row0, nrows)])
                    _striped(t, _drain)

    return functools.partial(
        pl.kernel,
        out_type=jax.ShapeDtypeStruct((num_slices, _N, _F), dtype),
        mesh=_mesh,
        scratch_types=[
            pltpu.VMEM_SHARED((_N, _F), dtype),
            pltpu.VMEM((_WIN, _K), jnp.int32),
            pltpu.VMEM((_WIN, _K), jnp.int32),
            pltpu.VMEM((_K, _F), dtype),
            pltpu.VMEM((_K, _F), dtype),
            pltpu.SemaphoreType.DMA,
            pltpu.SemaphoreType.DMA,
        ],
    )(body)


_agg_k2 = _make_agg(2, jnp.float32)
_agg_k4 = _make_agg(4, jnp.float32)


def _prep_body(x_ref, d0_ref, d1_ref, dinv_ref, xs_ref):
    deg = (d0_ref[...] + d1_ref[...]).reshape(_BN) + 1.0
    di = lax.rsqrt(deg)
    dinv_ref[...] = di.reshape(1, 1, _BN)
    xsb = x_ref[...] * di[:, None]
    xs_ref[0] = xsb[:, :_F]
    xs_ref[1] = xsb[:, _F:]


def _prep(x, d0, d1):
    return pl.pallas_call(
        _prep_body,
        grid=(_N // _BN,),
        in_specs=[
            pl.BlockSpec((_BN, _DIN), lambda i: (i, 0)),
            pl.BlockSpec((1, 1, _BN), lambda i: (i, 0, 0)),
            pl.BlockSpec((1, 1, _BN), lambda i: (i, 0, 0)),
        ],
        out_specs=[
            pl.BlockSpec((1, 1, _BN), lambda i: (i, 0, 0)),
            pl.BlockSpec((2, _BN, _F), lambda i: (0, i, 0)),
        ],
        out_shape=[
            jax.ShapeDtypeStruct((_N // _BN, 1, _BN), jnp.float32),
            jax.ShapeDtypeStruct((2, _N, _F), jnp.float32),
        ],
    )(x, d0, d1)


def _layer_body(num_slices, scale_out, agg_ref, dinv_ref, w_ref, b_ref, out_ref):
    di = dinv_ref[...].reshape(_BN)
    h = jnp.zeros((_BN, _DH), jnp.float32)
    for s in range(num_slices):
        a = (agg_ref[s] * di[:, None]).astype(jnp.bfloat16)
        h = h + jnp.dot(a, w_ref[s], preferred_element_type=jnp.float32)
    h = jnp.maximum(h + b_ref[...][None, :], 0.0)
    if scale_out:
        h = h * di[:, None]
        for s in range(_DH // _F):
            out_ref[s] = h[:, s * _F:(s + 1) * _F]
    else:
        out_ref[...] = h.astype(jnp.bfloat16)


def _layer(agg, dinv, w, b, scale_out):
    num_slices = w.shape[0]
    if scale_out:
        out_spec = pl.BlockSpec((_DH // _F, _BN, _F), lambda i: (0, i, 0))
        out_shape = jax.ShapeDtypeStruct((_DH // _F, _N, _F), jnp.float32)
    else:
        out_spec = pl.BlockSpec((_BN, _DH), lambda i: (i, 0))
        out_shape = jax.ShapeDtypeStruct((_N, _DH), jnp.bfloat16)
    return pl.pallas_call(
        functools.partial(_layer_body, num_slices, scale_out),
        grid=(_N // _BN,),
        in_specs=[
            pl.BlockSpec((num_slices, _BN, _F), lambda i: (0, i, 0)),
            pl.BlockSpec((1, 1, _BN), lambda i: (i, 0, 0)),
            pl.BlockSpec((num_slices, _F, _DH), lambda i: (0, 0, 0)),
            pl.BlockSpec((_DH,), lambda i: (0,)),
        ],
        out_specs=out_spec,
        out_shape=out_shape,
    )(agg, dinv, w, b)


def _pool_body(h2_ref, batch_ref, wout_ref, bout_ref, out_ref, acc, cnt):
    i = pl.program_id(0)

    @pl.when(i == 0)
    def _init():
        acc[...] = jnp.zeros_like(acc)
        cnt[...] = jnp.zeros_like(cnt)

    bt = batch_ref[...].reshape(_BN)
    gid = lax.broadcasted_iota(jnp.int32, (_G, _BN), 0)
    oh = (bt[None, :] == gid).astype(jnp.bfloat16)
    acc[...] += jnp.dot(oh, h2_ref[...], preferred_element_type=jnp.float32)
    cnt[...] += jnp.sum(oh.astype(jnp.float32), axis=1, keepdims=True)

    @pl.when(i == _N // _BN - 1)
    def _fin():
        pooled = (acc[...] / jnp.maximum(cnt[...], 1.0)).astype(jnp.bfloat16)
        out_ref[...] = (jnp.dot(pooled, wout_ref[...],
                                preferred_element_type=jnp.float32)
                        + bout_ref[...][None, :])


def _pool(h2, batch3, wout, bout):
    return pl.pallas_call(
        _pool_body,
        grid=(_N // _BN,),
        in_specs=[
            pl.BlockSpec((_BN, _DH), lambda i: (i, 0)),
            pl.BlockSpec((1, 1, _BN), lambda i: (i, 0, 0)),
            pl.BlockSpec((_DH, _DOUT), lambda i: (0, 0)),
            pl.BlockSpec((_DOUT,), lambda i: (0,)),
        ],
        out_specs=pl.BlockSpec((_G, _DOUT), lambda i: (0, 0)),
        out_shape=jax.ShapeDtypeStruct((_G, _DOUT), jnp.float32),
        scratch_shapes=[
            pltpu.VMEM((_G, _DH), jnp.float32),
            pltpu.VMEM((_G, 1), jnp.float32),
        ],
    )(h2, batch3, wout, bout)


def kernel(x, edge_index, batch, W1, b1, W2, b2, Wout, bout):
    src = edge_index[0]
    dst = edge_index[1]
    src2 = src.reshape(_E // _K, _K)
    dst2 = dst.reshape(_E // _K, _K)
    dst2h = dst.reshape(_E // _KH, _KH)
    batch3 = batch.reshape(_N // _BN, 1, _BN)

    deg0, deg1 = _hist(dst2h)
    d0 = deg0.reshape(_N // _BN, 1, _BN)
    d1 = deg1.reshape(_N // _BN, 1, _BN)
    dinv, xs = _prep(x, d0, d1)
    agg1 = _agg_k2(xs, src2, dst2)
    h1s = _layer(agg1, dinv, W1.reshape(2, _F, _DH).astype(jnp.bfloat16),
                 b1, scale_out=True)
    agg2 = _agg_k4(h1s, src2, dst2)
    h2 = _layer(agg2, dinv, W2.reshape(4, _F, _DH).astype(jnp.bfloat16),
                b2, scale_out=False)
    return _pool(h2, batch3, Wout.astype(jnp.bfloat16), bout)


# speedup vs baseline: 1.2960x; 1.0211x over previous
import functools

import jax
import jax.numpy as jnp
from jax import lax
from jax.experimental import pallas as pl
from jax.experimental.pallas import tpu as pltpu
from jax.experimental.pallas import tpu_sc as plsc

_N = 10000
_E = 160000
_G = 64
_DIN = 256
_DH = 512
_DOUT = 128

_F = 128
_NT = 16
_NC = 2
_S0 = 624
_SL = _N - (_NT - 1) * _S0
_K = 125
_RPT = (_E // _K) // _NT
_WIN = _RPT // 2
_KH = 125
_RPW = (_E // _KH) // (_NC * _NT)
_BN = 1000

_mesh = plsc.VectorSubcoreMesh(core_axis_name="c", subcore_axis_name="s")


def _striped(t, fn):
    @pl.when(t < _NT - 1)
    def _main():
        fn(t * _S0, _S0)

    @pl.when(t == _NT - 1)
    def _last():
        fn((_NT - 1) * _S0, _SL)


def _hist_body(dst2_hbm, out0_hbm, out1_hbm, hist, zbuf, obuf, hidx):
    c = lax.axis_index("c")
    t = lax.axis_index("s")

    def _zb(i, carry):
        zbuf[pl.ds(i * 16, 16)] = jnp.zeros((16,), jnp.float32)
        return carry

    lax.fori_loop(0, _SL // 16, _zb, 0)

    def _ob(i, carry):
        obuf[pl.ds(i * 16, 16)] = jnp.ones((16,), jnp.float32)
        return carry

    lax.fori_loop(0, 8, _ob, 0)

    def _zero(row0, nrows):
        pltpu.sync_copy(zbuf.at[pl.ds(0, nrows)], hist.at[pl.ds(row0, nrows)])

    _striped(t, _zero)
    plsc.subcore_barrier()

    w = c * _NT + t
    pltpu.sync_copy(dst2_hbm.at[pl.ds(w * _RPW, _RPW)], hidx)

    def _body(j, carry):
        pltpu.sync_copy(obuf.at[pl.ds(0, _KH)], hist.at[hidx.at[j]], add=True)
        return carry

    lax.fori_loop(0, _RPW, _body, 0)
    plsc.subcore_barrier()

    @pl.when(c == 0)
    def _d0():
        def _drain(row0, nrows):
            pltpu.sync_copy(hist.at[pl.ds(row0, nrows)],
                            zbuf.at[pl.ds(0, nrows)])
            pltpu.sync_copy(zbuf.at[pl.ds(0, nrows)],
                            out0_hbm.at[pl.ds(row0, nrows)])
        _striped(t, _drain)

    @pl.when(c == 1)
    def _d1():
        def _drain(row0, nrows):
            pltpu.sync_copy(hist.at[pl.ds(row0, nrows)],
                            zbuf.at[pl.ds(0, nrows)])
            pltpu.sync_copy(zbuf.at[pl.ds(0, nrows)],
                            out1_hbm.at[pl.ds(row0, nrows)])
        _striped(t, _drain)


_hist = functools.partial(
    pl.kernel,
    out_type=[jax.ShapeDtypeStruct((_N,), jnp.float32),
              jax.ShapeDtypeStruct((_N,), jnp.float32)],
    mesh=_mesh,
    scratch_types=[
        pltpu.VMEM_SHARED((_N,), jnp.float32),
        pltpu.VMEM((_SL,), jnp.float32),
        pltpu.VMEM((128,), jnp.float32),
        pltpu.VMEM((_RPW, _KH), jnp.int32),
    ],
)(_hist_body)


def _make_agg(num_slices, dtype):
    spc = num_slices // _NC

    def body(xs_hbm, src2_hbm, dst2_hbm, out_hbm,
             acc, sidx, didx, rb0, rb1, sem0, sem1):
        c = lax.axis_index("c")
        t = lax.axis_index("s")
        bufs = (rb0, rb1)
        sems = (sem0, sem1)

        for cv in range(_NC):
            @pl.when(c == cv)
            def _core():
                for j in range(spc):
                    s = cv * spc + j
                    tbl = xs_hbm.at[s]

                    def _init(row0, nrows):
                        pltpu.sync_copy(tbl.at[pl.ds(row0, nrows)],
                                        acc.at[pl.ds(row0, nrows)])
                    _striped(t, _init)
                    plsc.subcore_barrier()

                    def _gstart(r, b):
                        pltpu.make_async_copy(tbl.at[sidx.at[r]],
                                              bufs[b], sems[b]).start()

                    def _gwait(b):
                        pltpu.make_async_copy(tbl.at[sidx.at[0]],
                                              bufs[b], sems[b]).wait()

                    for g in range(_RPT // _WIN):
                        base = t * _RPT + g * _WIN
                        pltpu.sync_copy(src2_hbm.at[pl.ds(base, _WIN)], sidx)
                        pltpu.sync_copy(dst2_hbm.at[pl.ds(base, _WIN)], didx)
                        _gstart(0, 0)

                        def _pair(p, carry):
                            r0 = 2 * p
                            r1 = r0 + 1

                            _gstart(r1, 1)
                            _gwait(0)
                            pltpu.sync_copy(rb0, acc.at[didx.at[r0]],
                                            add=True)

                            @pl.when(r0 + 2 < _WIN)
                            def _g0():
                                _gstart(r0 + 2, 0)

                            _gwait(1)
                            pltpu.sync_copy(rb1, acc.at[didx.at[r1]],
                                            add=True)
                            return carry

                        lax.fori_loop(0, _WIN // 2, _pair, 0)

                    plsc.subcore_barrier()

                    def _drain(row0, nrows):
                        pltpu.sync_copy(acc.at[pl.ds(row0, nrows)],
                                        out_hbm.at[s].at[pl.ds(row0, nrows)])
                    _striped(t, _drain)

    return functools.partial(
        pl.kernel,
        out_type=jax.ShapeDtypeStruct((num_slices, _N, _F), dtype),
        mesh=_mesh,
        scratch_types=[
            pltpu.VMEM_SHARED((_N, _F), dtype),
            pltpu.VMEM((_WIN, _K), jnp.int32),
            pltpu.VMEM((_WIN, _K), jnp.int32),
            pltpu.VMEM((_K, _F), dtype),
            pltpu.VMEM((_K, _F), dtype),
            pltpu.SemaphoreType.DMA,
            pltpu.SemaphoreType.DMA,
        ],
    )(body)


_agg_k2 = _make_agg(2, jnp.float32)
_agg_k4 = _make_agg(4, jnp.float32)


def _prep_body(x_ref, d0_ref, d1_ref, dinv_ref, xs_ref):
    deg = (d0_ref[...] + d1_ref[...]).reshape(_BN) + 1.0
    di = lax.rsqrt(deg)
    dinv_ref[...] = di.reshape(1, 1, _BN)
    xsb = x_ref[...] * di[:, None]
    xs_ref[0] = xsb[:, :_F]
    xs_ref[1] = xsb[:, _F:]


def _prep(x, d0, d1):
    return pl.pallas_call(
        _prep_body,
        grid=(_N // _BN,),
        in_specs=[
            pl.BlockSpec((_BN, _DIN), lambda i: (i, 0)),
            pl.BlockSpec((1, 1, _BN), lambda i: (i, 0, 0)),
            pl.BlockSpec((1, 1, _BN), lambda i: (i, 0, 0)),
        ],
        out_specs=[
            pl.BlockSpec((1, 1, _BN), lambda i: (i, 0, 0)),
            pl.BlockSpec((2, _BN, _F), lambda i: (0, i, 0)),
        ],
        out_shape=[
            jax.ShapeDtypeStruct((_N // _BN, 1, _BN), jnp.float32),
            jax.ShapeDtypeStruct((2, _N, _F), jnp.float32),
        ],
    )(x, d0, d1)


def _layer_body(num_slices, scale_out, agg_ref, dinv_ref, w_ref, b_ref, out_ref):
    di = dinv_ref[...].reshape(_BN)
    h = jnp.zeros((_BN, _DH), jnp.float32)
    for s in range(num_slices):
        a = (agg_ref[s] * di[:, None]).astype(jnp.bfloat16)
        h = h + jnp.dot(a, w_ref[s], preferred_element_type=jnp.float32)
    h = jnp.maximum(h + b_ref[...][None, :], 0.0)
    if scale_out:
        h = h * di[:, None]
        for s in range(_DH // _F):
            out_ref[s] = h[:, s * _F:(s + 1) * _F]
    else:
        out_ref[...] = h.astype(jnp.bfloat16)


def _layer(agg, dinv, w, b, scale_out):
    num_slices = w.shape[0]
    if scale_out:
        out_spec = pl.BlockSpec((_DH // _F, _BN, _F), lambda i: (0, i, 0))
        out_shape = jax.ShapeDtypeStruct((_DH // _F, _N, _F), jnp.float32)
    else:
        out_spec = pl.BlockSpec((_BN, _DH), lambda i: (i, 0))
        out_shape = jax.ShapeDtypeStruct((_N, _DH), jnp.bfloat16)
    return pl.pallas_call(
        functools.partial(_layer_body, num_slices, scale_out),
        grid=(_N // _BN,),
        in_specs=[
            pl.BlockSpec((num_slices, _BN, _F), lambda i: (0, i, 0)),
            pl.BlockSpec((1, 1, _BN), lambda i: (i, 0, 0)),
            pl.BlockSpec((num_slices, _F, _DH), lambda i: (0, 0, 0)),
            pl.BlockSpec((_DH,), lambda i: (0,)),
        ],
        out_specs=out_spec,
        out_shape=out_shape,
    )(agg, dinv, w, b)


def _l2pool_body(agg_ref, dinv_ref, batch_ref, w_ref, b_ref, wout_ref,
                 bout_ref, out_ref, acc, cnt):
    i = pl.program_id(0)

    @pl.when(i == 0)
    def _init():
        acc[...] = jnp.zeros_like(acc)
        cnt[...] = jnp.zeros_like(cnt)

    di = dinv_ref[...].reshape(_BN)
    h = jnp.zeros((_BN, _DH), jnp.float32)
    for s in range(4):
        a = (agg_ref[s] * di[:, None]).astype(jnp.bfloat16)
        h = h + jnp.dot(a, w_ref[s], preferred_element_type=jnp.float32)
    h = jnp.maximum(h + b_ref[...][None, :], 0.0)

    bt = batch_ref[...].reshape(_BN)
    gid = lax.broadcasted_iota(jnp.int32, (_G, _BN), 0)
    oh = (bt[None, :] == gid).astype(jnp.bfloat16)
    acc[...] += jnp.dot(oh, h.astype(jnp.bfloat16),
                        preferred_element_type=jnp.float32)
    cnt[...] += jnp.sum(oh.astype(jnp.float32), axis=1, keepdims=True)

    @pl.when(i == _N // _BN - 1)
    def _fin():
        pooled = (acc[...] / jnp.maximum(cnt[...], 1.0)).astype(jnp.bfloat16)
        out_ref[...] = (jnp.dot(pooled, wout_ref[...],
                                preferred_element_type=jnp.float32)
                        + bout_ref[...][None, :])


def _l2pool(agg, dinv, batch3, w, b, wout, bout):
    return pl.pallas_call(
        _l2pool_body,
        grid=(_N // _BN,),
        in_specs=[
            pl.BlockSpec((4, _BN, _F), lambda i: (0, i, 0)),
            pl.BlockSpec((1, 1, _BN), lambda i: (i, 0, 0)),
            pl.BlockSpec((1, 1, _BN), lambda i: (i, 0, 0)),
            pl.BlockSpec((4, _F, _DH), lambda i: (0, 0, 0)),
            pl.BlockSpec((_DH,), lambda i: (0,)),
            pl.BlockSpec((_DH, _DOUT), lambda i: (0, 0)),
            pl.BlockSpec((_DOUT,), lambda i: (0,)),
        ],
        out_specs=pl.BlockSpec((_G, _DOUT), lambda i: (0, 0)),
        out_shape=jax.ShapeDtypeStruct((_G, _DOUT), jnp.float32),
        scratch_shapes=[
            pltpu.VMEM((_G, _DH), jnp.float32),
            pltpu.VMEM((_G, 1), jnp.float32),
        ],
    )(agg, dinv, batch3, w, b, wout, bout)


def kernel(x, edge_index, batch, W1, b1, W2, b2, Wout, bout):
    src = edge_index[0]
    dst = edge_index[1]
    src2 = src.reshape(_E // _K, _K)
    dst2 = dst.reshape(_E // _K, _K)
    dst2h = dst.reshape(_E // _KH, _KH)
    batch3 = batch.reshape(_N // _BN, 1, _BN)

    deg0, deg1 = _hist(dst2h)
    d0 = deg0.reshape(_N // _BN, 1, _BN)
    d1 = deg1.reshape(_N // _BN, 1, _BN)
    dinv, xs = _prep(x, d0, d1)
    agg1 = _agg_k2(xs, src2, dst2)
    h1s = _layer(agg1, dinv, W1.reshape(2, _F, _DH).astype(jnp.bfloat16),
                 b1, scale_out=True)
    agg2 = _agg_k4(h1s, src2, dst2)
    return _l2pool(agg2, dinv, batch3,
                   W2.reshape(4, _F, _DH).astype(jnp.bfloat16), b2,
                   Wout.astype(jnp.bfloat16), bout)


# speedup vs baseline: 1.3028x; 1.0052x over previous
import functools

import jax
import jax.numpy as jnp
from jax import lax
from jax.experimental import pallas as pl
from jax.experimental.pallas import tpu as pltpu
from jax.experimental.pallas import tpu_sc as plsc

_N = 10000
_E = 160000
_G = 64
_DIN = 256
_DH = 512
_DOUT = 128

_F = 128
_NT = 16
_NC = 2
_S0 = 624
_SL = _N - (_NT - 1) * _S0
_K = 125
_RPT = (_E // _K) // _NT
_WIN = _RPT // 2
_KH = 125
_RPW = (_E // _KH) // (_NC * _NT)
_BN = 1000

_mesh = plsc.VectorSubcoreMesh(core_axis_name="c", subcore_axis_name="s")


def _striped(t, fn):
    @pl.when(t < _NT - 1)
    def _main():
        fn(t * _S0, _S0)

    @pl.when(t == _NT - 1)
    def _last():
        fn((_NT - 1) * _S0, _SL)


def _hist_body(dst2_hbm, out0_hbm, out1_hbm, hist, zbuf, obuf, hidx):
    c = lax.axis_index("c")
    t = lax.axis_index("s")

    def _zb(i, carry):
        zbuf[pl.ds(i * 16, 16)] = jnp.zeros((16,), jnp.float32)
        return carry

    lax.fori_loop(0, _SL // 16, _zb, 0)

    def _ob(i, carry):
        obuf[pl.ds(i * 16, 16)] = jnp.ones((16,), jnp.float32)
        return carry

    lax.fori_loop(0, 8, _ob, 0)

    def _zero(row0, nrows):
        pltpu.sync_copy(zbuf.at[pl.ds(0, nrows)], hist.at[pl.ds(row0, nrows)])

    _striped(t, _zero)
    plsc.subcore_barrier()

    w = c * _NT + t
    pltpu.sync_copy(dst2_hbm.at[pl.ds(w * _RPW, _RPW)], hidx)

    def _body(j, carry):
        pltpu.sync_copy(obuf.at[pl.ds(0, _KH)], hist.at[hidx.at[j]], add=True)
        return carry

    lax.fori_loop(0, _RPW, _body, 0)
    plsc.subcore_barrier()

    @pl.when(c == 0)
    def _d0():
        def _drain(row0, nrows):
            pltpu.sync_copy(hist.at[pl.ds(row0, nrows)],
                            zbuf.at[pl.ds(0, nrows)])
            pltpu.sync_copy(zbuf.at[pl.ds(0, nrows)],
                            out0_hbm.at[pl.ds(row0, nrows)])
        _striped(t, _drain)

    @pl.when(c == 1)
    def _d1():
        def _drain(row0, nrows):
            pltpu.sync_copy(hist.at[pl.ds(row0, nrows)],
                            zbuf.at[pl.ds(0, nrows)])
            pltpu.sync_copy(zbuf.at[pl.ds(0, nrows)],
                            out1_hbm.at[pl.ds(row0, nrows)])
        _striped(t, _drain)


_hist = functools.partial(
    pl.kernel,
    out_type=[jax.ShapeDtypeStruct((_N,), jnp.float32),
              jax.ShapeDtypeStruct((_N,), jnp.float32)],
    mesh=_mesh,
    scratch_types=[
        pltpu.VMEM_SHARED((_N,), jnp.float32),
        pltpu.VMEM((_SL,), jnp.float32),
        pltpu.VMEM((128,), jnp.float32),
        pltpu.VMEM((_RPW, _KH), jnp.int32),
    ],
)(_hist_body)


def _make_agg(num_slices, dtype):
    spc = num_slices // _NC

    def body(xs_hbm, src2_hbm, dst2_hbm, out_hbm,
             acc, sidx, didx, rb0, rb1, sem0, sem1):
        c = lax.axis_index("c")
        t = lax.axis_index("s")
        bufs = (rb0, rb1)
        sems = (sem0, sem1)

        for cv in range(_NC):
            @pl.when(c == cv)
            def _core():
                for j in range(spc):
                    s = cv * spc + j
                    tbl = xs_hbm.at[s]

                    def _init(row0, nrows):
                        pltpu.sync_copy(tbl.at[pl.ds(row0, nrows)],
                                        acc.at[pl.ds(row0, nrows)])
                    _striped(t, _init)
                    plsc.subcore_barrier()

                    def _gstart(r, b):
                        pltpu.make_async_copy(tbl.at[sidx.at[r]],
                                              bufs[b], sems[b]).start()

                    def _gwait(b):
                        pltpu.make_async_copy(tbl.at[sidx.at[0]],
                                              bufs[b], sems[b]).wait()

                    for g in range(_RPT // _WIN):
                        base = t * _RPT + g * _WIN
                        pltpu.sync_copy(src2_hbm.at[pl.ds(base, _WIN)], sidx)
                        pltpu.sync_copy(dst2_hbm.at[pl.ds(base, _WIN)], didx)
                        _gstart(0, 0)

                        def _pair(p, carry):
                            r0 = 2 * p
                            r1 = r0 + 1

                            _gstart(r1, 1)
                            _gwait(0)
                            pltpu.sync_copy(rb0, acc.at[didx.at[r0]],
                                            add=True)

                            @pl.when(r0 + 2 < _WIN)
                            def _g0():
                                _gstart(r0 + 2, 0)

                            _gwait(1)
                            pltpu.sync_copy(rb1, acc.at[didx.at[r1]],
                                            add=True)
                            return carry

                        lax.fori_loop(0, _WIN // 2, _pair, 0)

                    plsc.subcore_barrier()

                    def _drain(row0, nrows):
                        pltpu.sync_copy(acc.at[pl.ds(row0, nrows)],
                                        out_hbm.at[s].at[pl.ds(row0, nrows)])
                    _striped(t, _drain)

    return functools.partial(
        pl.kernel,
        out_type=jax.ShapeDtypeStruct((num_slices, _N, _F), dtype),
        mesh=_mesh,
        scratch_types=[
            pltpu.VMEM_SHARED((_N, _F), dtype),
            pltpu.VMEM((_WIN, _K), jnp.int32),
            pltpu.VMEM((_WIN, _K), jnp.int32),
            pltpu.VMEM((_K, _F), dtype),
            pltpu.VMEM((_K, _F), dtype),
            pltpu.SemaphoreType.DMA,
            pltpu.SemaphoreType.DMA,
        ],
    )(body)


_agg_k2 = _make_agg(2, jnp.float32)
_agg_k4 = _make_agg(4, jnp.float32)


def _prep_body(x_ref, d0_ref, d1_ref, dinv_ref, xs_ref):
    deg = (d0_ref[...] + d1_ref[...]).reshape(_BN) + 1.0
    di = lax.rsqrt(deg)
    dinv_ref[...] = di.reshape(1, 1, _BN)
    xsb = x_ref[...] * di[:, None]
    xs_ref[0] = xsb[:, :_F]
    xs_ref[1] = xsb[:, _F:]


def _prep(x, d0, d1):
    return pl.pallas_call(
        _prep_body,
        grid=(_N // _BN,),
        in_specs=[
            pl.BlockSpec((_BN, _DIN), lambda i: (i, 0)),
            pl.BlockSpec((1, 1, _BN), lambda i: (i, 0, 0)),
            pl.BlockSpec((1, 1, _BN), lambda i: (i, 0, 0)),
        ],
        out_specs=[
            pl.BlockSpec((1, 1, _BN), lambda i: (i, 0, 0)),
            pl.BlockSpec((2, _BN, _F), lambda i: (0, i, 0)),
        ],
        out_shape=[
            jax.ShapeDtypeStruct((_N // _BN, 1, _BN), jnp.float32),
            jax.ShapeDtypeStruct((2, _N, _F), jnp.float32),
        ],
    )(x, d0, d1)


def _layer_body(agg_ref, dinv_ref, w_ref, b_ref, out_ref):
    di = dinv_ref[...].reshape(_BN)
    h = jnp.zeros((_BN, _DH), jnp.float32)
    for s in range(2):
        a = (agg_ref[s] * di[:, None]).astype(jnp.bfloat16)
        h = h + jnp.dot(a, w_ref[s], preferred_element_type=jnp.float32)
    h = jnp.maximum(h + b_ref[...][None, :], 0.0)
    h = h * di[:, None]
    for s in range(_DH // _F):
        out_ref[s] = h[:, s * _F:(s + 1) * _F]


def _layer(agg, dinv, w, b):
    return pl.pallas_call(
        _layer_body,
        grid=(_N // _BN,),
        in_specs=[
            pl.BlockSpec((2, _BN, _F), lambda i: (0, i, 0)),
            pl.BlockSpec((1, 1, _BN), lambda i: (i, 0, 0)),
            pl.BlockSpec((2, _F, _DH), lambda i: (0, 0, 0)),
            pl.BlockSpec((_DH,), lambda i: (0,)),
        ],
        out_specs=pl.BlockSpec((_DH // _F, _BN, _F), lambda i: (0, i, 0)),
        out_shape=jax.ShapeDtypeStruct((_DH // _F, _N, _F), jnp.float32),
    )(agg, dinv, w, b)


def _l2pool_body(agg_ref, dinv_ref, batch_ref, w_ref, b_ref, wout_ref,
                 bout_ref, out_ref, acc, cnt):
    i = pl.program_id(0)

    @pl.when(i == 0)
    def _init():
        acc[...] = jnp.zeros_like(acc)
        cnt[...] = jnp.zeros_like(cnt)

    di = dinv_ref[...].reshape(_BN)
    h = jnp.zeros((_BN, _DH), jnp.float32)
    for s in range(4):
        a = (agg_ref[s] * di[:, None]).astype(jnp.bfloat16)
        h = h + jnp.dot(a, w_ref[s], preferred_element_type=jnp.float32)
    h = jnp.maximum(h + b_ref[...][None, :], 0.0)

    bt = batch_ref[...].reshape(_BN)
    gid = lax.broadcasted_iota(jnp.int32, (_G, _BN), 0)
    oh = (bt[None, :] == gid).astype(jnp.bfloat16)
    acc[...] += jnp.dot(oh, h.astype(jnp.bfloat16),
                        preferred_element_type=jnp.float32)
    cnt[...] += jnp.sum(oh.astype(jnp.float32), axis=1, keepdims=True)

    @pl.when(i == _N // _BN - 1)
    def _fin():
        pooled = (acc[...] / jnp.maximum(cnt[...], 1.0)).astype(jnp.bfloat16)
        out_ref[...] = (jnp.dot(pooled, wout_ref[...],
                                preferred_element_type=jnp.float32)
                        + bout_ref[...][None, :])


def _l2pool(agg, dinv, batch3, w, b, wout, bout):
    return pl.pallas_call(
        _l2pool_body,
        grid=(_N // _BN,),
        in_specs=[
            pl.BlockSpec((4, _BN, _F), lambda i: (0, i, 0)),
            pl.BlockSpec((1, 1, _BN), lambda i: (i, 0, 0)),
            pl.BlockSpec((1, 1, _BN), lambda i: (i, 0, 0)),
            pl.BlockSpec((4, _F, _DH), lambda i: (0, 0, 0)),
            pl.BlockSpec((_DH,), lambda i: (0,)),
            pl.BlockSpec((_DH, _DOUT), lambda i: (0, 0)),
            pl.BlockSpec((_DOUT,), lambda i: (0,)),
        ],
        out_specs=pl.BlockSpec((_G, _DOUT), lambda i: (0, 0)),
        out_shape=jax.ShapeDtypeStruct((_G, _DOUT), jnp.float32),
        scratch_shapes=[
            pltpu.VMEM((_G, _DH), jnp.float32),
            pltpu.VMEM((_G, 1), jnp.float32),
        ],
    )(agg, dinv, batch3, w, b, wout, bout)


def kernel(x, edge_index, batch, W1, b1, W2, b2, Wout, bout):
    src = edge_index[0]
    dst = edge_index[1]
    src2 = src.reshape(_E // _K, _K)
    dst2 = dst.reshape(_E // _K, _K)
    batch3 = batch.reshape(_N // _BN, 1, _BN)

    deg0, deg1 = _hist(dst2)
    d0 = deg0.reshape(_N // _BN, 1, _BN)
    d1 = deg1.reshape(_N // _BN, 1, _BN)
    dinv, xs = _prep(x, d0, d1)
    agg1 = _agg_k2(xs, src2, dst2)
    h1s = _layer(agg1, dinv, W1.reshape(2, _F, _DH).astype(jnp.bfloat16), b1)
    agg2 = _agg_k4(h1s, src2, dst2)
    return _l2pool(agg2, dinv, batch3,
                   W2.reshape(4, _F, _DH).astype(jnp.bfloat16), b2,
                   Wout.astype(jnp.bfloat16), bout)
